# baseline (speedup 1.0000x reference)
import functools

import jax
import jax.numpy as jnp
from jax import lax
from jax.experimental import pallas as pl
from jax.experimental.pallas import tpu as pltpu
from jax.experimental.pallas import tpu_sc as plsc

N = 10000
E = 320000
D = 128
H = 128
NC = 2
NS = 16
NW = NC * NS
CHUNK = 128
NCHUNKS = E // CHUNK
CPT = (NCHUNKS + NW - 1) // NW
CPT = 80
NCHUNKS_PAD = CPT * NW
EPAD = NCHUNKS_PAD * CHUNK
NBUF = 2
NPAD = 10240
RPT = NPAD // NS
EPS = 1e-5

_MESH = plsc.VectorSubcoreMesh(
    core_axis_name="c", subcore_axis_name="s", num_cores=NC, num_subcores=NS)


def _zero_fill(ref, nrows, ncols):
  z16 = jnp.zeros((16,), jnp.float32)

  def body(i, _):
    for j in range(ncols // 16):
      ref[i, pl.ds(j * 16, 16)] = z16
    return 0

  lax.fori_loop(0, nrows, body, 0)


@functools.partial(
    pl.kernel,
    out_type=jax.ShapeDtypeStruct((NC * NPAD, 16), jnp.float32),
    mesh=_MESH,
    scratch_types=[
        pltpu.VMEM_SHARED((NPAD, 16), jnp.float32),
        pltpu.VMEM((1, CHUNK), jnp.int32),
        pltpu.VMEM((CHUNK, 16), jnp.float32),
        pltpu.VMEM((RPT, 16), jnp.float32),
    ],
)
def _sc_degree(dst_hbm, out_hbm, hist, didx, ones, zeros):
  cid = lax.axis_index("c")
  sid = lax.axis_index("s")
  wid = sid * NC + cid

  one16 = jnp.ones((16,), jnp.float32)

  def fill_ones(i, _):
    ones[i, pl.ds(0, 16)] = one16
    return 0

  lax.fori_loop(0, CHUNK, fill_ones, 0)
  _zero_fill(zeros, RPT, 16)
  pltpu.sync_copy(zeros, hist.at[pl.ds(sid * RPT, RPT)])
  plsc.subcore_barrier()

  def body(it, _):
    g = it * NW + wid
    pltpu.sync_copy(dst_hbm.at[pl.ds(g * CHUNK, CHUNK)], didx.at[0])
    pltpu.sync_copy(ones, hist.at[didx.at[0]], add=True)
    return 0

  lax.fori_loop(0, CPT, body, 0)
  plsc.subcore_barrier()
  pltpu.sync_copy(hist.at[pl.ds(sid * RPT, RPT)],
                  out_hbm.at[pl.ds(cid * NPAD + sid * RPT, RPT)])


NI = 4


@functools.partial(
    pl.kernel,
    out_type=jax.ShapeDtypeStruct((NC * NPAD, H), jnp.float32),
    mesh=_MESH,
    scratch_types=[
        pltpu.VMEM_SHARED((NPAD, H), jnp.float32),
        pltpu.VMEM((NI, CHUNK), jnp.int32),
        pltpu.VMEM((NI, CHUNK), jnp.int32),
        pltpu.VMEM((CHUNK, H), jnp.float32),
        pltpu.VMEM((RPT // 20, H), jnp.float32),
    ] + [pltpu.SemaphoreType.DMA] * 11,
)
def _sc_aggregate(u_hbm, src_hbm, dst_hbm, out_hbm, acc, sibuf, dibuf,
                  rows, zeros, *sems):
  cid = lax.axis_index("c")
  sid = lax.axis_index("s")
  wid = sid * NC + cid
  ebase = wid * CPT * CHUNK
  isems = list(sems[0:4])
  dsems = list(sems[4:8])
  gsem = sems[8]
  ssem = sems[9]
  zsem = sems[10]

  zr = RPT // 20
  _zero_fill(zeros, zr, H)
  zdesc = []
  for k in range(20):
    zdesc.append(pltpu.async_copy(
        zeros, acc.at[pl.ds(sid * RPT + k * zr, zr)], zsem))
  for d in zdesc:
    d.wait()
  plsc.subcore_barrier()

  ides = [None] * CPT
  ides2 = [None] * CPT

  def _fetch(c):
    off = ebase + c * CHUNK
    ides[c] = pltpu.async_copy(src_hbm.at[pl.ds(off, CHUNK)],
                               sibuf.at[c % NI], isems[c % NI])
    ides2[c] = pltpu.async_copy(dst_hbm.at[pl.ds(off, CHUNK)],
                                dibuf.at[c % NI], dsems[c % NI])

  for c in range(min(NI, CPT)):
    _fetch(c)
  for c in range(CPT):
    ides[c].wait()
    ides2[c].wait()
    pltpu.async_copy(u_hbm.at[sibuf.at[c % NI]], rows, gsem).wait()
    pltpu.async_copy(rows, acc.at[dibuf.at[c % NI]], ssem, add=True).wait()
    if c + NI < CPT:
      _fetch(c + NI)
  plsc.subcore_barrier()
  pltpu.sync_copy(acc.at[pl.ds(sid * RPT, RPT)],
                  out_hbm.at[pl.ds(cid * NPAD + sid * RPT, RPT)])


def _tc_first_body(hist_ref, x_ref, w_ref, dinv_ref, u_ref):
  deg = hist_ref[pl.ds(0, N), 0:1] + hist_ref[pl.ds(NPAD, N), 0:1] + 1.0
  dinv = lax.rsqrt(deg)
  dinv_ref[...] = dinv
  u_ref[...] = jnp.dot(x_ref[...], w_ref[...],
                       preferred_element_type=jnp.float32) * dinv


def _tc_mid_body(part_ref, u_ref, dinv_ref, b_ref, g_ref, be_ref, w_ref,
                 unext_ref):
  dinv = dinv_ref[...]
  t = (part_ref[pl.ds(0, N), :] + part_ref[pl.ds(NPAD, N), :] + u_ref[...]) * dinv
  y = jnp.maximum(t + b_ref[...], 0.0)
  mu = jnp.mean(y, axis=0, keepdims=True)
  var = jnp.mean((y - mu) ** 2, axis=0, keepdims=True)
  z = (y - mu) * lax.rsqrt(var + EPS) * g_ref[...] + be_ref[...]
  unext_ref[...] = jnp.dot(z, w_ref[...],
                           preferred_element_type=jnp.float32) * dinv


def _tc_last_body(part_ref, u_ref, dinv_ref, b_ref, g_ref, be_ref, out_ref):
  dinv = dinv_ref[...]
  t = (part_ref[pl.ds(0, N), :] + part_ref[pl.ds(NPAD, N), :] + u_ref[...]) * dinv
  y = jnp.maximum(t + b_ref[...], 0.0)
  mu = jnp.mean(y, axis=0, keepdims=True)
  var = jnp.mean((y - mu) ** 2, axis=0, keepdims=True)
  out_ref[...] = (y - mu) * lax.rsqrt(var + EPS) * g_ref[...] + be_ref[...]


_tc_first = pl.pallas_call(
    _tc_first_body,
    out_shape=(jax.ShapeDtypeStruct((N, 1), jnp.float32),
               jax.ShapeDtypeStruct((N, H), jnp.float32)),
)

_tc_mid = pl.pallas_call(
    _tc_mid_body,
    out_shape=jax.ShapeDtypeStruct((N, H), jnp.float32),
)

_tc_last = pl.pallas_call(
    _tc_last_body,
    out_shape=jax.ShapeDtypeStruct((N, H), jnp.float32),
)


def kernel(x, edge_index, W1, b1, g1, be1, W2, b2, g2, be2, W3, b3, g3, be3,
           W4, b4, g4, be4):
  npad_e = EPAD - E
  src = jnp.concatenate(
      [edge_index[0], jnp.zeros((npad_e,), jnp.int32)])
  dst = jnp.concatenate(
      [edge_index[1],
       N + (jnp.arange(npad_e, dtype=jnp.int32) % (NPAD - N))])
  hist = _sc_degree(dst)
  dinv, u = _tc_first(hist, x, W1)

  params = [(b1, g1, be1), (b2, g2, be2), (b3, g3, be3), (b4, g4, be4)]
  nxt = [W2, W3, W4]
  for i in range(4):
    b, g, be = params[i]
    part = _sc_aggregate(u, src, dst)
    b2d = b.reshape(1, H)
    g2d = g.reshape(1, H)
    be2d = be.reshape(1, H)
    if i < 3:
      u = _tc_mid(part, u, dinv, b2d, g2d, be2d, nxt[i])
    else:
      u = _tc_last(part, u, dinv, b2d, g2d, be2d)
  return u


# speedup vs baseline: 1.1644x; 1.1644x over previous
import functools

import jax
import jax.numpy as jnp
from jax import lax
from jax.experimental import pallas as pl
from jax.experimental.pallas import tpu as pltpu
from jax.experimental.pallas import tpu_sc as plsc

N = 10000
E = 320000
D = 128
H = 128
NC = 2
NS = 16
NW = NC * NS
CHUNK = 128
NCHUNKS = E // CHUNK
CPT = (NCHUNKS + NW - 1) // NW
CPT = 80
NCHUNKS_PAD = CPT * NW
EPAD = NCHUNKS_PAD * CHUNK
NBUF = 2
NPAD = 10240
RPT = NPAD // NS
EPS = 1e-5

_MESH = plsc.VectorSubcoreMesh(
    core_axis_name="c", subcore_axis_name="s", num_cores=NC, num_subcores=NS)


def _zero_fill(ref, nrows, ncols):
  z16 = jnp.zeros((16,), jnp.float32)

  def body(i, _):
    for j in range(ncols // 16):
      ref[i, pl.ds(j * 16, 16)] = z16
    return 0

  lax.fori_loop(0, nrows, body, 0)


@functools.partial(
    pl.kernel,
    out_type=jax.ShapeDtypeStruct((NC * NPAD, 16), jnp.float32),
    mesh=_MESH,
    scratch_types=[
        pltpu.VMEM_SHARED((NPAD, 16), jnp.float32),
        pltpu.VMEM((1, CHUNK), jnp.int32),
        pltpu.VMEM((CHUNK, 16), jnp.float32),
        pltpu.VMEM((RPT, 16), jnp.float32),
    ],
)
def _sc_degree(dst_hbm, out_hbm, hist, didx, ones, zeros):
  cid = lax.axis_index("c")
  sid = lax.axis_index("s")
  wid = sid * NC + cid

  one16 = jnp.ones((16,), jnp.float32)

  def fill_ones(i, _):
    ones[i, pl.ds(0, 16)] = one16
    return 0

  lax.fori_loop(0, CHUNK, fill_ones, 0)
  _zero_fill(zeros, RPT, 16)
  pltpu.sync_copy(zeros, hist.at[pl.ds(sid * RPT, RPT)])
  plsc.subcore_barrier()

  def body(it, _):
    g = it * NW + wid
    pltpu.sync_copy(dst_hbm.at[pl.ds(g * CHUNK, CHUNK)], didx.at[0])
    pltpu.sync_copy(ones, hist.at[didx.at[0]], add=True)
    return 0

  lax.fori_loop(0, CPT, body, 0)
  plsc.subcore_barrier()
  pltpu.sync_copy(hist.at[pl.ds(sid * RPT, RPT)],
                  out_hbm.at[pl.ds(cid * NPAD + sid * RPT, RPT)])


NI = 4


@functools.partial(
    pl.kernel,
    out_type=jax.ShapeDtypeStruct((NC * NPAD, H), jnp.float32),
    mesh=_MESH,
    scratch_types=[
        pltpu.VMEM_SHARED((NPAD, H), jnp.float32),
        pltpu.VMEM((NI, CHUNK), jnp.int32),
        pltpu.VMEM((NI, CHUNK), jnp.int32),
        pltpu.VMEM((CHUNK, H), jnp.float32),
        pltpu.VMEM((CHUNK, H), jnp.float32),
        pltpu.VMEM((RPT // 20, H), jnp.float32),
    ] + [pltpu.SemaphoreType.DMA] * 13,
)
def _sc_aggregate(u_hbm, src_hbm, dst_hbm, out_hbm, acc, sibuf, dibuf,
                  rows0, rows1, zeros, *sems):
  cid = lax.axis_index("c")
  sid = lax.axis_index("s")
  wid = sid * NC + cid
  ebase = wid * CPT * CHUNK
  isems = list(sems[0:4])
  dsems = list(sems[4:8])
  gsems = list(sems[8:10])
  ssems = list(sems[10:12])
  zsem = sems[12]

  zr = RPT // 20
  _zero_fill(zeros, zr, H)
  zdesc = []
  for k in range(20):
    zdesc.append(pltpu.async_copy(
        zeros, acc.at[pl.ds(sid * RPT + k * zr, zr)], zsem))
  for d in zdesc:
    d.wait()
  plsc.subcore_barrier()

  rowbufs = [rows0, rows1]

  def fetch(c, b):
    off = ebase + c * CHUNK
    pltpu.async_copy(src_hbm.at[pl.ds(off, CHUNK)], sibuf.at[b], isems[b])
    pltpu.async_copy(dst_hbm.at[pl.ds(off, CHUNK)], dibuf.at[b], dsems[b])

  def wait_idx(b):
    pltpu.make_async_copy(src_hbm.at[pl.ds(0, CHUNK)], sibuf.at[b],
                          isems[b]).wait()
    pltpu.make_async_copy(dst_hbm.at[pl.ds(0, CHUNK)], dibuf.at[b],
                          dsems[b]).wait()

  def gather(b, r):
    pltpu.async_copy(u_hbm.at[sibuf.at[b]], rowbufs[r], gsems[r])

  def wait_gather(b, r):
    pltpu.make_async_copy(u_hbm.at[sibuf.at[b]], rowbufs[r],
                          gsems[r]).wait()

  def scatter(b, r):
    pltpu.async_copy(rowbufs[r], acc.at[dibuf.at[b]], ssems[r], add=True)

  def wait_scatter(b, r):
    pltpu.make_async_copy(rowbufs[r], acc.at[dibuf.at[b]], ssems[r]).wait()

  def step(c, b):
    wait_scatter((b + 2) % 4, b % 2)
    wait_idx(b)
    gather(b, b % 2)
    wait_gather((b + 3) % 4, (b + 1) % 2)
    scatter((b + 3) % 4, (b + 1) % 2)

    @pl.when(c + 2 < CPT)
    def _():
      fetch(c + 2, (b + 2) % 4)

  fetch(0, 0)
  fetch(1, 1)
  wait_idx(0)
  gather(0, 0)
  fetch(2, 2)
  wait_idx(1)
  gather(1, 1)
  wait_gather(0, 0)
  scatter(0, 0)
  fetch(3, 3)
  step(2, 2)
  step(3, 3)

  def group(o, _):
    for b in range(4):
      step(o * 4 + b, b)
    return 0

  lax.fori_loop(1, CPT // 4, group, 0)

  wait_gather(3, 1)
  scatter(3, 1)
  wait_scatter(2, 0)
  wait_scatter(3, 1)
  plsc.subcore_barrier()
  pltpu.sync_copy(acc.at[pl.ds(sid * RPT, RPT)],
                  out_hbm.at[pl.ds(cid * NPAD + sid * RPT, RPT)])


def _tc_first_body(hist_ref, x_ref, w_ref, dinv_ref, u_ref):
  deg = hist_ref[pl.ds(0, N), 0:1] + hist_ref[pl.ds(NPAD, N), 0:1] + 1.0
  dinv = lax.rsqrt(deg)
  dinv_ref[...] = dinv
  u_ref[...] = jnp.dot(x_ref[...], w_ref[...],
                       preferred_element_type=jnp.float32) * dinv


def _tc_mid_body(part_ref, u_ref, dinv_ref, b_ref, g_ref, be_ref, w_ref,
                 unext_ref):
  dinv = dinv_ref[...]
  t = (part_ref[pl.ds(0, N), :] + part_ref[pl.ds(NPAD, N), :] + u_ref[...]) * dinv
  y = jnp.maximum(t + b_ref[...], 0.0)
  mu = jnp.mean(y, axis=0, keepdims=True)
  var = jnp.mean((y - mu) ** 2, axis=0, keepdims=True)
  z = (y - mu) * lax.rsqrt(var + EPS) * g_ref[...] + be_ref[...]
  unext_ref[...] = jnp.dot(z, w_ref[...],
                           preferred_element_type=jnp.float32) * dinv


def _tc_last_body(part_ref, u_ref, dinv_ref, b_ref, g_ref, be_ref, out_ref):
  dinv = dinv_ref[...]
  t = (part_ref[pl.ds(0, N), :] + part_ref[pl.ds(NPAD, N), :] + u_ref[...]) * dinv
  y = jnp.maximum(t + b_ref[...], 0.0)
  mu = jnp.mean(y, axis=0, keepdims=True)
  var = jnp.mean((y - mu) ** 2, axis=0, keepdims=True)
  out_ref[...] = (y - mu) * lax.rsqrt(var + EPS) * g_ref[...] + be_ref[...]


_tc_first = pl.pallas_call(
    _tc_first_body,
    out_shape=(jax.ShapeDtypeStruct((N, 1), jnp.float32),
               jax.ShapeDtypeStruct((N, H), jnp.float32)),
)

_tc_mid = pl.pallas_call(
    _tc_mid_body,
    out_shape=jax.ShapeDtypeStruct((N, H), jnp.float32),
)

_tc_last = pl.pallas_call(
    _tc_last_body,
    out_shape=jax.ShapeDtypeStruct((N, H), jnp.float32),
)


def kernel(x, edge_index, W1, b1, g1, be1, W2, b2, g2, be2, W3, b3, g3, be3,
           W4, b4, g4, be4):
  npad_e = EPAD - E
  src = jnp.concatenate(
      [edge_index[0], jnp.zeros((npad_e,), jnp.int32)])
  dst = jnp.concatenate(
      [edge_index[1],
       N + (jnp.arange(npad_e, dtype=jnp.int32) % (NPAD - N))])
  hist = _sc_degree(dst)
  dinv, u = _tc_first(hist, x, W1)

  params = [(b1, g1, be1), (b2, g2, be2), (b3, g3, be3), (b4, g4, be4)]
  nxt = [W2, W3, W4]
  for i in range(4):
    b, g, be = params[i]
    part = _sc_aggregate(u, src, dst)
    b2d = b.reshape(1, H)
    g2d = g.reshape(1, H)
    be2d = be.reshape(1, H)
    if i < 3:
      u = _tc_mid(part, u, dinv, b2d, g2d, be2d, nxt[i])
    else:
      u = _tc_last(part, u, dinv, b2d, g2d, be2d)
  return u


# speedup vs baseline: 1.2337x; 1.0595x over previous
import functools

import jax
import jax.numpy as jnp
from jax import lax
from jax.experimental import pallas as pl
from jax.experimental.pallas import tpu as pltpu
from jax.experimental.pallas import tpu_sc as plsc

N = 10000
E = 320000
D = 128
H = 128
NC = 2
NS = 16
NW = NC * NS
CHUNK = 128
NCHUNKS = E // CHUNK
CPT = (NCHUNKS + NW - 1) // NW
CPT = 80
NCHUNKS_PAD = CPT * NW
EPAD = NCHUNKS_PAD * CHUNK
NBUF = 2
NPAD = 10240
RPT = NPAD // NS
EPS = 1e-5

_MESH = plsc.VectorSubcoreMesh(
    core_axis_name="c", subcore_axis_name="s", num_cores=NC, num_subcores=NS)


def _zero_fill(ref, nrows, ncols):
  z16 = jnp.zeros((16,), jnp.float32)

  def body(i, _):
    for j in range(ncols // 16):
      ref[i, pl.ds(j * 16, 16)] = z16
    return 0

  lax.fori_loop(0, nrows, body, 0)


@functools.partial(
    pl.kernel,
    out_type=jax.ShapeDtypeStruct((NC * NPAD, 16), jnp.float32),
    mesh=_MESH,
    scratch_types=[
        pltpu.VMEM_SHARED((NPAD, 16), jnp.float32),
        pltpu.VMEM((1, CHUNK), jnp.int32),
        pltpu.VMEM((CHUNK, 16), jnp.float32),
        pltpu.VMEM((RPT, 16), jnp.float32),
    ],
)
def _sc_degree(dst_hbm, out_hbm, hist, didx, ones, zeros):
  cid = lax.axis_index("c")
  sid = lax.axis_index("s")
  wid = sid * NC + cid

  one16 = jnp.ones((16,), jnp.float32)

  def fill_ones(i, _):
    ones[i, pl.ds(0, 16)] = one16
    return 0

  lax.fori_loop(0, CHUNK, fill_ones, 0)
  _zero_fill(zeros, RPT, 16)
  pltpu.sync_copy(zeros, hist.at[pl.ds(sid * RPT, RPT)])
  plsc.subcore_barrier()

  def body(it, _):
    g = it * NW + wid
    pltpu.sync_copy(dst_hbm.at[pl.ds(g * CHUNK, CHUNK)], didx.at[0])
    pltpu.sync_copy(ones, hist.at[didx.at[0]], add=True)
    return 0

  lax.fori_loop(0, CPT, body, 0)
  plsc.subcore_barrier()
  pltpu.sync_copy(hist.at[pl.ds(sid * RPT, RPT)],
                  out_hbm.at[pl.ds(cid * NPAD + sid * RPT, RPT)])


NI = 4


@functools.partial(
    pl.kernel,
    out_type=jax.ShapeDtypeStruct((NC * NPAD, H), jnp.float32),
    mesh=_MESH,
    scratch_types=[
        pltpu.VMEM_SHARED((NPAD, H), jnp.float32),
        pltpu.VMEM((NI, CHUNK), jnp.int32),
        pltpu.VMEM((NI, CHUNK), jnp.int32),
        pltpu.VMEM((CHUNK, H), jnp.float32),
        pltpu.VMEM((CHUNK, H), jnp.float32),
        pltpu.VMEM((RPT // 20, H), jnp.float32),
    ] + [pltpu.SemaphoreType.DMA] * 13,
)
def _sc_aggregate(u_hbm, src_hbm, dst_hbm, out_hbm, acc, sibuf, dibuf,
                  rows0, rows1, zeros, *sems):
  cid = lax.axis_index("c")
  sid = lax.axis_index("s")
  wid = sid * NC + cid
  isems = list(sems[0:4])
  dsems = list(sems[4:8])
  gsems = list(sems[8:10])
  ssems = list(sems[10:12])
  zsem = sems[12]

  zr = RPT // 20
  _zero_fill(zeros, zr, H)
  zdesc = []
  for k in range(20):
    zdesc.append(pltpu.async_copy(
        zeros, acc.at[pl.ds(sid * RPT + k * zr, zr)], zsem))
  for d in zdesc:
    d.wait()
  plsc.subcore_barrier()

  rowbufs = [rows0, rows1]

  def fetch(c, b):
    off = (c * NW + wid) * CHUNK
    pltpu.async_copy(src_hbm.at[pl.ds(off, CHUNK)], sibuf.at[b], isems[b])
    pltpu.async_copy(dst_hbm.at[pl.ds(off, CHUNK)], dibuf.at[b], dsems[b])

  def wait_idx(b):
    pltpu.make_async_copy(src_hbm.at[pl.ds(0, CHUNK)], sibuf.at[b],
                          isems[b]).wait()
    pltpu.make_async_copy(dst_hbm.at[pl.ds(0, CHUNK)], dibuf.at[b],
                          dsems[b]).wait()

  def gather(b, r):
    pltpu.async_copy(u_hbm.at[sibuf.at[b]], rowbufs[r], gsems[r])

  def wait_gather(b, r):
    pltpu.make_async_copy(u_hbm.at[sibuf.at[b]], rowbufs[r],
                          gsems[r]).wait()

  def scatter(b, r):
    pltpu.async_copy(rowbufs[r], acc.at[dibuf.at[b]], ssems[r], add=True)

  def wait_scatter(b, r):
    pltpu.make_async_copy(rowbufs[r], acc.at[dibuf.at[b]], ssems[r]).wait()

  def step(c, b):
    wait_scatter((b + 2) % 4, b % 2)
    wait_idx(b)
    gather(b, b % 2)
    wait_gather((b + 3) % 4, (b + 1) % 2)
    scatter((b + 3) % 4, (b + 1) % 2)

    @pl.when(c + 2 < CPT)
    def _():
      fetch(c + 2, (b + 2) % 4)

  fetch(0, 0)
  fetch(1, 1)
  wait_idx(0)
  gather(0, 0)
  fetch(2, 2)
  wait_idx(1)
  gather(1, 1)
  wait_gather(0, 0)
  scatter(0, 0)
  fetch(3, 3)
  step(2, 2)
  step(3, 3)

  def group(o, _):
    for b in range(4):
      step(o * 4 + b, b)
    return 0

  lax.fori_loop(1, CPT // 4, group, 0)

  wait_gather(3, 1)
  scatter(3, 1)
  wait_scatter(2, 0)
  wait_scatter(3, 1)
  plsc.subcore_barrier()
  pltpu.sync_copy(acc.at[pl.ds(sid * RPT, RPT)],
                  out_hbm.at[pl.ds(cid * NPAD + sid * RPT, RPT)])


def _tc_first_body(hist_ref, x_ref, w_ref, dinv_ref, u_ref):
  deg = hist_ref[pl.ds(0, N), 0:1] + hist_ref[pl.ds(NPAD, N), 0:1] + 1.0
  dinv = lax.rsqrt(deg)
  dinv_ref[...] = dinv
  u_ref[...] = jnp.dot(x_ref[...], w_ref[...],
                       preferred_element_type=jnp.float32) * dinv


def _tc_mid_body(part_ref, u_ref, dinv_ref, b_ref, g_ref, be_ref, w_ref,
                 unext_ref):
  dinv = dinv_ref[...]
  t = (part_ref[pl.ds(0, N), :] + part_ref[pl.ds(NPAD, N), :] + u_ref[...]) * dinv
  y = jnp.maximum(t + b_ref[...], 0.0)
  mu = jnp.mean(y, axis=0, keepdims=True)
  var = jnp.mean((y - mu) ** 2, axis=0, keepdims=True)
  z = (y - mu) * lax.rsqrt(var + EPS) * g_ref[...] + be_ref[...]
  unext_ref[...] = jnp.dot(z, w_ref[...],
                           preferred_element_type=jnp.float32) * dinv


def _tc_last_body(part_ref, u_ref, dinv_ref, b_ref, g_ref, be_ref, out_ref):
  dinv = dinv_ref[...]
  t = (part_ref[pl.ds(0, N), :] + part_ref[pl.ds(NPAD, N), :] + u_ref[...]) * dinv
  y = jnp.maximum(t + b_ref[...], 0.0)
  mu = jnp.mean(y, axis=0, keepdims=True)
  var = jnp.mean((y - mu) ** 2, axis=0, keepdims=True)
  out_ref[...] = (y - mu) * lax.rsqrt(var + EPS) * g_ref[...] + be_ref[...]


_tc_first = pl.pallas_call(
    _tc_first_body,
    out_shape=(jax.ShapeDtypeStruct((N, 1), jnp.float32),
               jax.ShapeDtypeStruct((N, H), jnp.float32)),
)

_tc_mid = pl.pallas_call(
    _tc_mid_body,
    out_shape=jax.ShapeDtypeStruct((N, H), jnp.float32),
)

_tc_last = pl.pallas_call(
    _tc_last_body,
    out_shape=jax.ShapeDtypeStruct((N, H), jnp.float32),
)


def kernel(x, edge_index, W1, b1, g1, be1, W2, b2, g2, be2, W3, b3, g3, be3,
           W4, b4, g4, be4):
  npad_e = EPAD - E
  src = jnp.concatenate(
      [edge_index[0], jnp.zeros((npad_e,), jnp.int32)])
  dst = jnp.concatenate(
      [edge_index[1],
       N + (jnp.arange(npad_e, dtype=jnp.int32) % (NPAD - N))])
  hist = _sc_degree(dst)
  dinv, u = _tc_first(hist, x, W1)

  params = [(b1, g1, be1), (b2, g2, be2), (b3, g3, be3), (b4, g4, be4)]
  nxt = [W2, W3, W4]
  for i in range(4):
    b, g, be = params[i]
    part = _sc_aggregate(u, src, dst)
    b2d = b.reshape(1, H)
    g2d = g.reshape(1, H)
    be2d = be.reshape(1, H)
    if i < 3:
      u = _tc_mid(part, u, dinv, b2d, g2d, be2d, nxt[i])
    else:
      u = _tc_last(part, u, dinv, b2d, g2d, be2d)
  return u
